# Initial kernel scaffold; baseline (speedup 1.0000x reference)
import jax
import jax.numpy as jnp
from jax.experimental import pallas as pl


def kernel(x, edge_index, edge_weight, W1, b1, W2, b2):
    raise NotImplementedError("write your pallas kernel here")


# speedup vs baseline: 15.9637x; 15.9637x over previous
import dataclasses
import functools

import jax
import jax.numpy as jnp
from jax import lax
from jax.experimental import pallas as pl
from jax.experimental.pallas import tpu as pltpu
from jax.experimental.pallas import tpu_sc as plsc

N = 10000
F_IN = 128
HID = 64
C = 40
CP = 48
E = 320000

NC = 2
NS = 16
NW = NC * NS
EPW = 10240
EP = NW * EPW
PAD = EP - E

ECH = 4096
NCH = EP // NC // ECH

_mesh = plsc.VectorSubcoreMesh(
    core_axis_name="c", subcore_axis_name="s", num_cores=NC, num_subcores=NS
)

_sc_params = pltpu.CompilerParams()
if "needs_layout_passes" in pltpu.CompilerParams.__dataclass_fields__:
    _sc_params = dataclasses.replace(_sc_params, needs_layout_passes=False)
if "use_tc_tiling_on_sc" in pltpu.CompilerParams.__dataclass_fields__:
    _sc_params = dataclasses.replace(_sc_params, use_tc_tiling_on_sc=False)


def _deg_body(dst_hbm, w_hbm, out_hbm, deg_v, dst_v, w_v, sem):
    cid = lax.axis_index("c")
    sid = lax.axis_index("s")
    wid = sid * NC + cid
    c1 = pltpu.async_copy(dst_hbm.at[wid], dst_v, sem)
    c2 = pltpu.async_copy(w_hbm.at[wid], w_v, sem)
    zero16 = jnp.zeros((16,), jnp.float32)

    @pl.loop(0, N, step=16)
    def _(i):
        deg_v[pl.ds(i, 16)] = zero16

    c1.wait()
    c2.wait()

    @pl.loop(0, EPW, step=16)
    def _(i):
        idx = dst_v[pl.ds(i, 16)]
        val = w_v[pl.ds(i, 16)]
        plsc.addupdate_scatter(deg_v, [idx], val)

    pltpu.async_copy(deg_v, out_hbm.at[wid], sem).wait()


_deg_kernel = functools.partial(
    pl.kernel,
    _deg_body,
    out_type=jax.ShapeDtypeStruct((NW, N), jnp.float32),
    mesh=_mesh,
    scratch_types=[
        pltpu.VMEM((N,), jnp.float32),
        pltpu.VMEM((EPW,), jnp.int32),
        pltpu.VMEM((EPW,), jnp.float32),
        pltpu.SemaphoreType.DMA,
    ],
    compiler_params=_sc_params,
)()


def _make_msg_kernel(F):
    FPW = F // NS

    def body(hn_hbm, src_hbm, dst_hbm, w_hbm, out_hbm,
             hn_v, acc_v, s0, s1, d0, d1, w0, w1,
             hsem, e0, e1):
        sbuf = [s0, s1]
        dbuf = [d0, d1]
        wbuf = [w0, w1]
        esem = [e0, e1]
        cid = lax.axis_index("c")
        sid = lax.axis_index("s")

        hcp = pltpu.async_copy(hn_hbm.at[pl.ds(sid * FPW, FPW)], hn_v, hsem)
        for b in range(2):
            ch = cid * NCH + b
            pltpu.async_copy(src_hbm.at[ch], sbuf[b], esem[b])
            pltpu.async_copy(dst_hbm.at[ch], dbuf[b], esem[b])
            pltpu.async_copy(w_hbm.at[ch], wbuf[b], esem[b])

        zero16 = jnp.zeros((16,), jnp.float32)

        @pl.loop(0, N, step=16)
        def _(i):
            for f in range(FPW):
                acc_v[f, pl.ds(i, 16)] = zero16

        hcp.wait()

        @pl.loop(0, NCH, step=2)
        def _(c0):
            for b in range(2):
                ch = c0 + b
                hch = cid * NCH + ch
                pltpu.make_async_copy(src_hbm.at[hch], sbuf[b], esem[b]).wait()
                pltpu.make_async_copy(dst_hbm.at[hch], dbuf[b], esem[b]).wait()
                pltpu.make_async_copy(w_hbm.at[hch], wbuf[b], esem[b]).wait()

                @pl.loop(0, ECH, step=16, unroll=4)
                def _(i):
                    s16 = sbuf[b][pl.ds(i, 16)]
                    d16 = dbuf[b][pl.ds(i, 16)]
                    w16 = wbuf[b][pl.ds(i, 16)]
                    for f in range(FPW):
                        fi = jnp.full((16,), f, jnp.int32)
                        v = plsc.load_gather(hn_v, [fi, s16])
                        plsc.addupdate_scatter(acc_v, [fi, d16], v * w16)

                @pl.when(ch + 2 < NCH)
                def _():
                    nch = cid * NCH + ch + 2
                    pltpu.async_copy(src_hbm.at[nch], sbuf[b], esem[b])
                    pltpu.async_copy(dst_hbm.at[nch], dbuf[b], esem[b])
                    pltpu.async_copy(w_hbm.at[nch], wbuf[b], esem[b])

        pltpu.async_copy(acc_v, out_hbm.at[cid * NS + sid], hsem).wait()

    return functools.partial(
        pl.kernel,
        body,
        out_type=jax.ShapeDtypeStruct((NW, FPW, N), jnp.float32),
        mesh=_mesh,
        scratch_types=(
            [pltpu.VMEM((FPW, N), jnp.float32),
             pltpu.VMEM((FPW, N), jnp.float32)]
            + [pltpu.VMEM((ECH,), jnp.int32)] * 4
            + [pltpu.VMEM((ECH,), jnp.float32)] * 2
            + [pltpu.SemaphoreType.DMA] * 3
        ),
        compiler_params=_sc_params,
    )()


_msg_hid = _make_msg_kernel(HID)
_msg_cp = _make_msg_kernel(CP)


def _mm1_body(w1_ref, x_ref, o_ref):
    o_ref[...] = lax.dot_general(
        w1_ref[...], x_ref[...], (((0,), (1,)), ((), ())),
        preferred_element_type=jnp.float32,
    )


def _prep_body(degp_ref, h_ref, hn_ref, dinv_ref):
    deg = jnp.sum(degp_ref[...], axis=0, keepdims=True) + 1.0
    dinv = jnp.where(deg > 0, lax.rsqrt(jnp.where(deg > 0, deg, 1.0)), 0.0)
    hn_ref[...] = h_ref[...] * dinv
    dinv_ref[...] = dinv


def _mid_body(acc_ref, hn_ref, dinv_ref, b1_ref, w2_ref, hn2_ref):
    dinv = dinv_ref[...]
    s = (acc_ref[0] + acc_ref[1] + hn_ref[...]) * dinv + b1_ref[...]
    z = jnp.maximum(s, 0.0)
    hn2_ref[...] = lax.dot_general(
        w2_ref[...], z, (((0,), (0,)), ((), ())),
        preferred_element_type=jnp.float32,
    ) * dinv


def _fin_body(acc_ref, hn2_ref, dinv_ref, b2_ref, o_ref):
    s = (acc_ref[0] + acc_ref[1] + hn2_ref[...]) * dinv_ref[...] + b2_ref[...]
    t = s[:C, :]
    m = jnp.max(t, axis=0, keepdims=True)
    lse = jnp.log(jnp.sum(jnp.exp(t - m), axis=0, keepdims=True)) + m
    o_ref[...] = t - lse


def kernel(x, edge_index, edge_weight, W1, b1, W2, b2):
    x = x.astype(jnp.float32)
    src = edge_index[0].astype(jnp.int32)
    dst = edge_index[1].astype(jnp.int32)
    w = edge_weight.astype(jnp.float32)

    pad_idx = jnp.arange(PAD, dtype=jnp.int32)
    srcp = jnp.concatenate([src, pad_idx])
    dstp = jnp.concatenate([dst, pad_idx])
    wp = jnp.concatenate([w, jnp.zeros((PAD,), jnp.float32)])
    dst2 = dstp.reshape(NW, EPW)
    w2d = wp.reshape(NW, EPW)
    src_e = srcp.reshape(NC * NCH, ECH)
    dst_e = dstp.reshape(NC * NCH, ECH)
    w_e = wp.reshape(NC * NCH, ECH)

    W2p = jnp.concatenate([W2, jnp.zeros((HID, CP - C), jnp.float32)], axis=1)
    b1c = b1.reshape(HID, 1)
    b2c = jnp.concatenate([b2, jnp.zeros((CP - C,), jnp.float32)]).reshape(CP, 1)

    degp = _deg_kernel(dst2, w2d)
    h1t = pl.pallas_call(
        _mm1_body,
        out_shape=jax.ShapeDtypeStruct((HID, N), jnp.float32),
    )(W1, x)

    hn1t, dinv = pl.pallas_call(
        _prep_body,
        out_shape=(
            jax.ShapeDtypeStruct((HID, N), jnp.float32),
            jax.ShapeDtypeStruct((1, N), jnp.float32),
        ),
    )(degp, h1t)

    acc1 = _msg_hid(hn1t, src_e, dst_e, w_e).reshape(NC, HID, N)

    hn2t = pl.pallas_call(
        _mid_body,
        out_shape=jax.ShapeDtypeStruct((CP, N), jnp.float32),
    )(acc1, hn1t, dinv, b1c, W2p)

    acc2 = _msg_cp(hn2t, src_e, dst_e, w_e).reshape(NC, CP, N)

    outt = pl.pallas_call(
        _fin_body,
        out_shape=jax.ShapeDtypeStruct((C, N), jnp.float32),
    )(acc2, hn2t, dinv, b2c)
    return outt.T


# speedup vs baseline: 36.0437x; 2.2579x over previous
import dataclasses
import functools

import jax
import jax.numpy as jnp
from jax import lax
from jax.experimental import pallas as pl
from jax.experimental.pallas import tpu as pltpu
from jax.experimental.pallas import tpu_sc as plsc

N = 10000
F_IN = 128
HID = 64
C = 40
CP = 48
E = 320000

NC = 2
NS = 16
NW = NC * NS
EPW = 10240
EP = NW * EPW
PAD = EP - E

ECH = 4096
NCH = EP // NC // ECH

_mesh = plsc.VectorSubcoreMesh(
    core_axis_name="c", subcore_axis_name="s", num_cores=NC, num_subcores=NS
)

_sc_params = pltpu.CompilerParams()
if "needs_layout_passes" in pltpu.CompilerParams.__dataclass_fields__:
    _sc_params = dataclasses.replace(_sc_params, needs_layout_passes=False)
if "use_tc_tiling_on_sc" in pltpu.CompilerParams.__dataclass_fields__:
    _sc_params = dataclasses.replace(_sc_params, use_tc_tiling_on_sc=False)


def _deg_body(dst_hbm, w_hbm, out_hbm, deg_v, dst_v, w_v, sem):
    cid = lax.axis_index("c")
    sid = lax.axis_index("s")
    wid = sid * NC + cid
    c1 = pltpu.async_copy(dst_hbm.at[wid], dst_v, sem)
    c2 = pltpu.async_copy(w_hbm.at[wid], w_v, sem)
    zero16 = jnp.zeros((16,), jnp.float32)

    @pl.loop(0, N, step=16)
    def _(i):
        deg_v[pl.ds(i, 16)] = zero16

    c1.wait()
    c2.wait()

    @pl.loop(0, EPW, step=16)
    def _(i):
        idx = dst_v[pl.ds(i, 16)]
        val = w_v[pl.ds(i, 16)]
        plsc.addupdate_scatter(deg_v, [idx], val)

    pltpu.async_copy(deg_v, out_hbm.at[wid], sem).wait()


_deg_kernel = functools.partial(
    pl.kernel,
    _deg_body,
    out_type=jax.ShapeDtypeStruct((NW, N), jnp.float32),
    mesh=_mesh,
    scratch_types=[
        pltpu.VMEM((N,), jnp.float32),
        pltpu.VMEM((EPW,), jnp.int32),
        pltpu.VMEM((EPW,), jnp.float32),
        pltpu.SemaphoreType.DMA,
    ],
    compiler_params=_sc_params,
)()


def _make_msg_kernel(F):
    FPW = F // NS

    def body(hn_hbm, src_hbm, dst_hbm, w_hbm, out_hbm,
             hn_v, acc_v, s0, s1, d0, d1, w0, w1,
             hsem, e0, e1):
        sbuf = [s0, s1]
        dbuf = [d0, d1]
        wbuf = [w0, w1]
        esem = [e0, e1]
        cid = lax.axis_index("c")
        sid = lax.axis_index("s")

        hcp = pltpu.async_copy(hn_hbm.at[pl.ds(sid * FPW, FPW)], hn_v, hsem)
        for b in range(2):
            ch = cid * NCH + b
            pltpu.async_copy(src_hbm.at[ch], sbuf[b], esem[b])
            pltpu.async_copy(dst_hbm.at[ch], dbuf[b], esem[b])
            pltpu.async_copy(w_hbm.at[ch], wbuf[b], esem[b])

        zero16 = jnp.zeros((16,), jnp.float32)

        @pl.loop(0, N, step=16)
        def _(i):
            for f in range(FPW):
                acc_v[f, pl.ds(i, 16)] = zero16

        hcp.wait()

        @pl.loop(0, NCH, step=2)
        def _(c0):
            for b in range(2):
                ch = c0 + b
                hch = cid * NCH + ch
                pltpu.make_async_copy(src_hbm.at[hch], sbuf[b], esem[b]).wait()
                pltpu.make_async_copy(dst_hbm.at[hch], dbuf[b], esem[b]).wait()
                pltpu.make_async_copy(w_hbm.at[hch], wbuf[b], esem[b]).wait()

                @plsc.parallel_loop(0, ECH, step=16, unroll=4)
                def _(i):
                    s16 = sbuf[b][pl.ds(i, 16)]
                    d16 = dbuf[b][pl.ds(i, 16)]
                    w16 = wbuf[b][pl.ds(i, 16)]
                    for f in range(FPW):
                        fi = jnp.full((16,), f, jnp.int32)
                        v = plsc.load_gather(hn_v, [fi, s16])
                        plsc.addupdate_scatter(acc_v, [fi, d16], v * w16)

                @pl.when(ch + 2 < NCH)
                def _():
                    nch = cid * NCH + ch + 2
                    pltpu.async_copy(src_hbm.at[nch], sbuf[b], esem[b])
                    pltpu.async_copy(dst_hbm.at[nch], dbuf[b], esem[b])
                    pltpu.async_copy(w_hbm.at[nch], wbuf[b], esem[b])

        pltpu.async_copy(acc_v, out_hbm.at[cid * NS + sid], hsem).wait()

    return functools.partial(
        pl.kernel,
        body,
        out_type=jax.ShapeDtypeStruct((NW, FPW, N), jnp.float32),
        mesh=_mesh,
        scratch_types=(
            [pltpu.VMEM((FPW, N), jnp.float32),
             pltpu.VMEM((FPW, N), jnp.float32)]
            + [pltpu.VMEM((ECH,), jnp.int32)] * 4
            + [pltpu.VMEM((ECH,), jnp.float32)] * 2
            + [pltpu.SemaphoreType.DMA] * 3
        ),
        compiler_params=_sc_params,
    )()


_msg_hid = _make_msg_kernel(HID)
_msg_cp = _make_msg_kernel(CP)


def _mm1_body(w1_ref, x_ref, o_ref):
    o_ref[...] = lax.dot_general(
        w1_ref[...], x_ref[...], (((0,), (1,)), ((), ())),
        preferred_element_type=jnp.float32,
    )


def _prep_body(degp_ref, h_ref, hn_ref, dinv_ref):
    deg = jnp.sum(degp_ref[...], axis=0, keepdims=True) + 1.0
    dinv = jnp.where(deg > 0, lax.rsqrt(jnp.where(deg > 0, deg, 1.0)), 0.0)
    hn_ref[...] = h_ref[...] * dinv
    dinv_ref[...] = dinv


def _mid_body(acc_ref, hn_ref, dinv_ref, b1_ref, w2_ref, hn2_ref):
    dinv = dinv_ref[...]
    s = (acc_ref[0] + acc_ref[1] + hn_ref[...]) * dinv + b1_ref[...]
    z = jnp.maximum(s, 0.0)
    hn2_ref[...] = lax.dot_general(
        w2_ref[...], z, (((0,), (0,)), ((), ())),
        preferred_element_type=jnp.float32,
    ) * dinv


def _fin_body(acc_ref, hn2_ref, dinv_ref, b2_ref, o_ref):
    s = (acc_ref[0] + acc_ref[1] + hn2_ref[...]) * dinv_ref[...] + b2_ref[...]
    t = s[:C, :]
    m = jnp.max(t, axis=0, keepdims=True)
    lse = jnp.log(jnp.sum(jnp.exp(t - m), axis=0, keepdims=True)) + m
    o_ref[...] = t - lse


def kernel(x, edge_index, edge_weight, W1, b1, W2, b2):
    x = x.astype(jnp.float32)
    src = edge_index[0].astype(jnp.int32)
    dst = edge_index[1].astype(jnp.int32)
    w = edge_weight.astype(jnp.float32)

    pad_idx = jnp.arange(PAD, dtype=jnp.int32)
    srcp = jnp.concatenate([src, pad_idx])
    dstp = jnp.concatenate([dst, pad_idx])
    wp = jnp.concatenate([w, jnp.zeros((PAD,), jnp.float32)])
    dst2 = dstp.reshape(NW, EPW)
    w2d = wp.reshape(NW, EPW)
    src_e = srcp.reshape(NC * NCH, ECH)
    dst_e = dstp.reshape(NC * NCH, ECH)
    w_e = wp.reshape(NC * NCH, ECH)

    W2p = jnp.concatenate([W2, jnp.zeros((HID, CP - C), jnp.float32)], axis=1)
    b1c = b1.reshape(HID, 1)
    b2c = jnp.concatenate([b2, jnp.zeros((CP - C,), jnp.float32)]).reshape(CP, 1)

    degp = _deg_kernel(dst2, w2d)
    h1t = pl.pallas_call(
        _mm1_body,
        out_shape=jax.ShapeDtypeStruct((HID, N), jnp.float32),
    )(W1, x)

    hn1t, dinv = pl.pallas_call(
        _prep_body,
        out_shape=(
            jax.ShapeDtypeStruct((HID, N), jnp.float32),
            jax.ShapeDtypeStruct((1, N), jnp.float32),
        ),
    )(degp, h1t)

    acc1 = _msg_hid(hn1t, src_e, dst_e, w_e).reshape(NC, HID, N)

    hn2t = pl.pallas_call(
        _mid_body,
        out_shape=jax.ShapeDtypeStruct((CP, N), jnp.float32),
    )(acc1, hn1t, dinv, b1c, W2p)

    acc2 = _msg_cp(hn2t, src_e, dst_e, w_e).reshape(NC, CP, N)

    outt = pl.pallas_call(
        _fin_body,
        out_shape=jax.ShapeDtypeStruct((C, N), jnp.float32),
    )(acc2, hn2t, dinv, b2c)
    return outt.T
